# Initial kernel scaffold; baseline (speedup 1.0000x reference)
import functools

import jax
import jax.numpy as jnp
from jax import lax
from jax.experimental import pallas as pl
from jax.experimental.pallas import tpu as pltpu
from jax.experimental.pallas import tpu_sc as plsc

F32 = jnp.float32
STEPS = 3
TAU_THRESH = 0.05
N = 10000
E = 320000
H = 128
AW = H + 16
NC, NS = 2, 16
NW = NC * NS
EPW = E // NW
C = 80
CHUNKS = EPW // C
RPT = N // NS
ZR = 125


def _sc_edge_kernel():
    mesh = plsc.VectorSubcoreMesh(core_axis_name="c", subcore_axis_name="s")

    @functools.partial(
        pl.kernel,
        out_type=jax.ShapeDtypeStruct((NC, N, AW), F32),
        mesh=mesh,
        scratch_types=[
            pltpu.VMEM((C,), jnp.int32),
            pltpu.VMEM((C,), jnp.int32),
            pltpu.VMEM((C, H), F32),
            pltpu.VMEM((C, H), F32),
            pltpu.VMEM((C, AW), F32),
            pltpu.VMEM((ZR, AW), F32),
            pltpu.VMEM_SHARED((N, AW), F32),
            pltpu.SemaphoreType.DMA,
            pltpu.SemaphoreType.DMA,
        ],
    )
    def edge_kernel(h_hbm, row_hbm, col_hbm, out_hbm,
                    ri, ci, hi, hj, ob, zb, acc, sem1, sem2):
        c = lax.axis_index("c")
        s = lax.axis_index("s")
        wid = s * NC + c
        base = wid * EPW

        zero16 = jnp.zeros((16,), F32)

        @pl.loop(0, ZR)
        def _zero_rows(r):
            for k in range(AW // 16):
                zb[r, pl.ds(16 * k, 16)] = zero16

        for j in range(RPT // ZR):
            pltpu.sync_copy(zb, acc.at[pl.ds(s * RPT + j * ZR, ZR)])
        plsc.subcore_barrier()

        @pl.loop(0, CHUNKS)
        def _chunk(t):
            off = base + t * C
            pltpu.sync_copy(row_hbm.at[pl.ds(off, C)], ri)
            pltpu.sync_copy(col_hbm.at[pl.ds(off, C)], ci)
            cp1 = pltpu.async_copy(h_hbm.at[ri], hi, sem1)
            cp2 = pltpu.async_copy(h_hbm.at[ci], hj, sem2)
            cp1.wait()
            cp2.wait()

            @pl.loop(0, C)
            def _edge(e):
                s1 = jnp.zeros((16,), F32)
                s2 = jnp.zeros((16,), F32)
                for k in range(H // 16):
                    a = hi[e, pl.ds(16 * k, 16)]
                    b = hj[e, pl.ds(16 * k, 16)]
                    d = jnp.abs(a - b)
                    ob[e, pl.ds(16 * k, 16)] = d
                    s1 = s1 + d
                    s2 = s2 + d * d
                rs1 = jnp.sum(s1)
                rs2 = jnp.sum(s2)
                vi = (rs2 - rs1 * rs1 * (1.0 / H)) * (1.0 / (H - 1))
                lane = lax.iota(jnp.int32, 16)
                tail = jnp.where(lane == 0, vi,
                                 jnp.where(lane == 1, 1.0, 0.0))
                ob[e, pl.ds(H, 16)] = tail

            pltpu.sync_copy(ob, acc.at[ri], add=True)

        plsc.subcore_barrier()
        pltpu.sync_copy(acc.at[pl.ds(s * RPT, RPT)],
                        out_hbm.at[c, pl.ds(s * RPT, RPT)])

    return edge_kernel


_EDGE_KERNEL = _sc_edge_kernel()


def _dot_t(a, w):
    return lax.dot_general(a, w, (((1,), (1,)), ((), ())),
                           preferred_element_type=F32)


BN = 1000


def _pre_body(x_ref, w_ref, b_ref, o_ref):
    o_ref[...] = jnp.maximum(
        _dot_t(x_ref[...], w_ref[...]) + b_ref[...], 0.0)


def _step_core(p0, p1, h, wih, whh, bih, bhh, tp):
    agg = p0 + p1
    diff = agg[:, :H]
    vm = agg[:, H:H + 1] / (agg[:, H + 1:H + 2] + 1e-6)
    t = jnp.maximum(vm * tp[0:1, :] + tp[1:2, :], 0.0)
    tau = jax.nn.sigmoid(
        jnp.sum(t * tp[2:3, :], axis=1, keepdims=True) + tp[3, 0])
    alpha = jax.nn.sigmoid(TAU_THRESH - tau)
    gi = _dot_t(diff, wih) + bih
    gh = _dot_t(h, whh) + bhh
    r = jax.nn.sigmoid(gi[:, :H] + gh[:, :H])
    z = jax.nn.sigmoid(gi[:, H:2 * H] + gh[:, H:2 * H])
    n = jnp.tanh(gi[:, 2 * H:] + r * gh[:, 2 * H:])
    nh = (1.0 - z) * n + z * h
    return alpha * nh + (1.0 - alpha) * h


def _step_body(p0_ref, p1_ref, h_ref, wih_ref, whh_ref, bih_ref, bhh_ref,
               tp_ref, o_ref):
    o_ref[...] = _step_core(p0_ref[...], p1_ref[...], h_ref[...],
                            wih_ref[...], whh_ref[...], bih_ref[...],
                            bhh_ref[...], tp_ref[...])


def _step_final_body(p0_ref, p1_ref, h_ref, wih_ref, whh_ref, bih_ref,
                     bhh_ref, tp_ref, wout_ref, bout_ref, o_ref):
    hn = _step_core(p0_ref[...], p1_ref[...], h_ref[...],
                    wih_ref[...], whh_ref[...], bih_ref[...],
                    bhh_ref[...], tp_ref[...])
    o_ref[...] = _dot_t(hn, wout_ref[...]) + bout_ref[...]


def _blk(shape):
    return pl.BlockSpec(shape, lambda i: (0,) * len(shape))


def _nblk(width):
    return pl.BlockSpec((BN, width), lambda i: (i, 0))


_PRE_CALL = pl.pallas_call(
    _pre_body,
    grid=(N // BN,),
    in_specs=[_nblk(H), _blk((H, H)), _blk((1, H))],
    out_specs=_nblk(H),
    out_shape=jax.ShapeDtypeStruct((N, H), F32),
)

_STEP_CALL = pl.pallas_call(
    _step_body,
    grid=(N // BN,),
    in_specs=[_nblk(AW), _nblk(AW), _nblk(H),
              _blk((3 * H, H)), _blk((3 * H, H)),
              _blk((1, 3 * H)), _blk((1, 3 * H)), _blk((4, 16))],
    out_specs=_nblk(H),
    out_shape=jax.ShapeDtypeStruct((N, H), F32),
)

_STEP_FINAL_CALL = pl.pallas_call(
    _step_final_body,
    grid=(N // BN,),
    in_specs=[_nblk(AW), _nblk(AW), _nblk(H),
              _blk((3 * H, H)), _blk((3 * H, H)),
              _blk((1, 3 * H)), _blk((1, 3 * H)), _blk((4, 16)),
              _blk((H, H)), _blk((1, H))],
    out_specs=_nblk(H),
    out_shape=jax.ShapeDtypeStruct((N, H), F32),
)


def kernel(x, edge_index, W_in, b_in, W_tau1, b_tau1, W_tau2, b_tau2,
           W_ih, W_hh, b_ih, b_hh, W_out, b_out):
    row = edge_index[0]
    col = edge_index[1]
    bih = b_ih.reshape(1, 3 * H)
    bhh = b_hh.reshape(1, 3 * H)
    tp = jnp.stack([W_tau1[:, 0], b_tau1, W_tau2[0],
                    jnp.pad(b_tau2, (0, 15))])
    h = _PRE_CALL(x, W_in, b_in.reshape(1, H))
    for step in range(STEPS):
        parts = _EDGE_KERNEL(h, row, col)
        if step < STEPS - 1:
            h = _STEP_CALL(parts[0], parts[1], h, W_ih, W_hh, bih, bhh, tp)
        else:
            h = _STEP_FINAL_CALL(parts[0], parts[1], h, W_ih, W_hh, bih,
                                 bhh, tp, W_out, b_out.reshape(1, H))
    return h


# speedup vs baseline: 2.5655x; 2.5655x over previous
import functools

import jax
import jax.numpy as jnp
from jax import lax
from jax.experimental import pallas as pl
from jax.experimental.pallas import tpu as pltpu
from jax.experimental.pallas import tpu_sc as plsc

F32 = jnp.float32
STEPS = 3
TAU_THRESH = 0.05
N = 10000
E = 320000
H = 128
AW = H + 16
NC, NS = 2, 16
NW = NC * NS
EPW = E // NW
C = 80
CHUNKS = EPW // C
NP = 10240
RPT = NP // NS


def _sc_edge_kernel():
    mesh = plsc.VectorSubcoreMesh(core_axis_name="c", subcore_axis_name="s",
                                  num_cores=NC, num_subcores=NS)

    @functools.partial(
        pl.kernel,
        out_type=jax.ShapeDtypeStruct((NC, NP, AW), F32),
        mesh=mesh,
        compiler_params=pltpu.CompilerParams(needs_layout_passes=False,
                                             use_tc_tiling_on_sc=False),
        scratch_types=[
            pltpu.VMEM((C,), jnp.int32),
            pltpu.VMEM((C,), jnp.int32),
            pltpu.VMEM((C, H), F32),
            pltpu.VMEM((C, H), F32),
            pltpu.VMEM((C, AW), F32),
            pltpu.VMEM_SHARED((NP, AW), F32),
            pltpu.SemaphoreType.DMA,
            pltpu.SemaphoreType.DMA,
        ],
    )
    def edge_kernel(h_hbm, row_hbm, col_hbm, out_hbm,
                    ri, ci, hi, hj, ob, acc, sem1, sem2):
        c = lax.axis_index("c")
        s = lax.axis_index("s")
        wid = s * NC + c
        base = wid * EPW

        zero16 = jnp.zeros((16,), F32)

        @pl.loop(0, C)
        def _zero_rows(r):
            for k in range(AW // 16):
                ob[r, pl.ds(16 * k, 16)] = zero16

        for j in range(RPT // C):
            pltpu.sync_copy(ob, acc.at[pl.ds(s * RPT + j * C, C)])
        plsc.subcore_barrier()

        @pl.loop(0, CHUNKS)
        def _chunk(t):
            off = base + t * C
            pltpu.sync_copy(row_hbm.at[pl.ds(off, C)], ri)
            pltpu.sync_copy(col_hbm.at[pl.ds(off, C)], ci)
            cp1 = pltpu.async_copy(h_hbm.at[ri], hi, sem1)
            cp2 = pltpu.async_copy(h_hbm.at[ci], hj, sem2)
            cp1.wait()
            cp2.wait()

            @pl.loop(0, C)
            def _edge(e):
                s1 = jnp.zeros((16,), F32)
                s2 = jnp.zeros((16,), F32)
                for k in range(H // 16):
                    a = hi[e, pl.ds(16 * k, 16)]
                    b = hj[e, pl.ds(16 * k, 16)]
                    d = jnp.abs(a - b)
                    ob[e, pl.ds(16 * k, 16)] = d
                    s1 = s1 + d
                    s2 = s2 + d * d
                rs1 = jnp.sum(s1)
                rs2 = jnp.sum(s2)
                vi = (rs2 - rs1 * rs1 * (1.0 / H)) * (1.0 / (H - 1))
                lane = lax.iota(jnp.int32, 16)
                tail = jnp.where(lane == 0, vi,
                                 jnp.where(lane == 1, 1.0, 0.0))
                ob[e, pl.ds(H, 16)] = tail

            pltpu.sync_copy(ob, acc.at[ri], add=True)

        plsc.subcore_barrier()
        pltpu.sync_copy(acc.at[pl.ds(s * RPT, RPT)],
                        out_hbm.at[c, pl.ds(s * RPT, RPT)])

    return edge_kernel


_EDGE_KERNEL = _sc_edge_kernel()


def _dot_t(a, w):
    return lax.dot_general(a, w, (((1,), (1,)), ((), ())),
                           preferred_element_type=F32)


BN = 1000


def _pre_body(x_ref, w_ref, b_ref, o_ref):
    o_ref[...] = jnp.maximum(
        _dot_t(x_ref[...], w_ref[...]) + b_ref[...], 0.0)


def _step_core(p0, p1, h, wih, whh, bih, bhh, tp):
    agg = p0 + p1
    diff = agg[:, :H]
    vm = agg[:, H:H + 1] / (agg[:, H + 1:H + 2] + 1e-6)
    t = jnp.maximum(vm * tp[0:1, :] + tp[1:2, :], 0.0)
    tau = jax.nn.sigmoid(
        jnp.sum(t * tp[2:3, :], axis=1, keepdims=True) + tp[3, 0])
    alpha = jax.nn.sigmoid(TAU_THRESH - tau)
    gi = _dot_t(diff, wih) + bih
    gh = _dot_t(h, whh) + bhh
    r = jax.nn.sigmoid(gi[:, :H] + gh[:, :H])
    z = jax.nn.sigmoid(gi[:, H:2 * H] + gh[:, H:2 * H])
    n = jnp.tanh(gi[:, 2 * H:] + r * gh[:, 2 * H:])
    nh = (1.0 - z) * n + z * h
    return alpha * nh + (1.0 - alpha) * h


def _step_body(p0_ref, p1_ref, h_ref, wih_ref, whh_ref, bih_ref, bhh_ref,
               tp_ref, o_ref):
    o_ref[...] = _step_core(p0_ref[...], p1_ref[...], h_ref[...],
                            wih_ref[...], whh_ref[...], bih_ref[...],
                            bhh_ref[...], tp_ref[...])


def _step_final_body(p0_ref, p1_ref, h_ref, wih_ref, whh_ref, bih_ref,
                     bhh_ref, tp_ref, wout_ref, bout_ref, o_ref):
    hn = _step_core(p0_ref[...], p1_ref[...], h_ref[...],
                    wih_ref[...], whh_ref[...], bih_ref[...],
                    bhh_ref[...], tp_ref[...])
    o_ref[...] = _dot_t(hn, wout_ref[...]) + bout_ref[...]


def _blk(shape):
    return pl.BlockSpec(shape, lambda i: (0,) * len(shape))


def _nblk(width):
    return pl.BlockSpec((BN, width), lambda i: (i, 0))


_PRE_CALL = pl.pallas_call(
    _pre_body,
    grid=(N // BN,),
    in_specs=[_nblk(H), _blk((H, H)), _blk((1, H))],
    out_specs=_nblk(H),
    out_shape=jax.ShapeDtypeStruct((N, H), F32),
)

_STEP_CALL = pl.pallas_call(
    _step_body,
    grid=(N // BN,),
    in_specs=[_nblk(AW), _nblk(AW), _nblk(H),
              _blk((3 * H, H)), _blk((3 * H, H)),
              _blk((1, 3 * H)), _blk((1, 3 * H)), _blk((4, 16))],
    out_specs=_nblk(H),
    out_shape=jax.ShapeDtypeStruct((N, H), F32),
)

_STEP_FINAL_CALL = pl.pallas_call(
    _step_final_body,
    grid=(N // BN,),
    in_specs=[_nblk(AW), _nblk(AW), _nblk(H),
              _blk((3 * H, H)), _blk((3 * H, H)),
              _blk((1, 3 * H)), _blk((1, 3 * H)), _blk((4, 16)),
              _blk((H, H)), _blk((1, H))],
    out_specs=_nblk(H),
    out_shape=jax.ShapeDtypeStruct((N, H), F32),
)


def kernel(x, edge_index, W_in, b_in, W_tau1, b_tau1, W_tau2, b_tau2,
           W_ih, W_hh, b_ih, b_hh, W_out, b_out):
    row = edge_index[0]
    col = edge_index[1]
    bih = b_ih.reshape(1, 3 * H)
    bhh = b_hh.reshape(1, 3 * H)
    tp = jnp.stack([W_tau1[:, 0], b_tau1, W_tau2[0],
                    jnp.pad(b_tau2, (0, 15))])
    h = _PRE_CALL(x, W_in, b_in.reshape(1, H))
    for step in range(STEPS):
        parts = _EDGE_KERNEL(h, row, col)
        p0 = parts[0, :N]
        p1 = parts[1, :N]
        if step < STEPS - 1:
            h = _STEP_CALL(p0, p1, h, W_ih, W_hh, bih, bhh, tp)
        else:
            h = _STEP_FINAL_CALL(p0, p1, h, W_ih, W_hh, bih,
                                 bhh, tp, W_out, b_out.reshape(1, H))
    return h


# speedup vs baseline: 4.2663x; 1.6629x over previous
import functools

import jax
import jax.numpy as jnp
from jax import lax
from jax.experimental import pallas as pl
from jax.experimental.pallas import tpu as pltpu
from jax.experimental.pallas import tpu_sc as plsc

F32 = jnp.float32
STEPS = 3
TAU_THRESH = 0.05
N = 10000
E = 320000
H = 128
AW = H + 16
NC, NS = 2, 16
NW = NC * NS
EPW = E // NW
C = 40
BCH = 50
BLK = EPW // (BCH * C)
NP = 10240
RPT = NP // NS


def _sc_edge_kernel():
    mesh = plsc.VectorSubcoreMesh(core_axis_name="c", subcore_axis_name="s",
                                  num_cores=NC, num_subcores=NS)

    @functools.partial(
        pl.kernel,
        out_type=jax.ShapeDtypeStruct((NC, NP, AW), F32),
        mesh=mesh,
        compiler_params=pltpu.CompilerParams(needs_layout_passes=False,
                                             use_tc_tiling_on_sc=False),
        scratch_types=[
            pltpu.VMEM((BCH, C), jnp.int32),
            pltpu.VMEM((BCH, C), jnp.int32),
            pltpu.VMEM((C, H), F32),
            pltpu.VMEM((C, H), F32),
            pltpu.VMEM((C, H), F32),
            pltpu.VMEM((C, H), F32),
            pltpu.VMEM((C, AW), F32),
            pltpu.VMEM((C, AW), F32),
            pltpu.VMEM((16, 48), F32),
            pltpu.VMEM((16, 48), F32),
            pltpu.VMEM_SHARED((NP, AW), F32),
            pltpu.SemaphoreType.DMA,
            pltpu.SemaphoreType.DMA,
            pltpu.SemaphoreType.DMA,
            pltpu.SemaphoreType.DMA,
        ],
    )
    def edge_kernel(h_hbm, row_hbm, col_hbm, zeros_hbm, out_hbm,
                    rb, cb, hi0, hi1, hj0, hj1, ob0, ob1, s1t, s2t, acc,
                    sg0, sg1, ss0, ss1):
        c = lax.axis_index("c")
        s = lax.axis_index("s")
        wid = s * NC + c
        HI = (hi0, hi1)
        HJ = (hj0, hj1)
        OB = (ob0, ob1)
        SG = (sg0, sg1)
        SS = (ss0, ss1)

        iota = lax.iota(jnp.int32, 16)
        ones16 = jnp.full((16,), 1.0, F32)
        col_vi = jnp.full((16,), H, jnp.int32)
        col_ct = jnp.full((16,), H + 1, jnp.int32)
        halfmask = iota < 8

        pltpu.sync_copy(zeros_hbm.at[pl.ds(s * RPT, RPT)],
                        acc.at[pl.ds(s * RPT, RPT)])
        plsc.subcore_barrier()

        def issue_gathers(b, ch):
            cpi = pltpu.async_copy(h_hbm.at[rb.at[ch]], HI[b], SG[b])
            cpj = pltpu.async_copy(h_hbm.at[cb.at[ch]], HJ[b], SG[b])
            return cpi, cpj

        def wait_gathers(b, ch):
            pltpu.make_async_copy(h_hbm.at[rb.at[ch]], HI[b], SG[b]).wait()
            pltpu.make_async_copy(h_hbm.at[cb.at[ch]], HJ[b], SG[b]).wait()

        def wait_scatter(b, ch):
            pltpu.make_async_copy(OB[b], acc.at[rb.at[ch]], SS[b]).wait()

        def compute_chunk(b):
            hi, hj, ob = HI[b], HJ[b], OB[b]

            @pl.loop(0, C)
            def _edge(e):
                s1 = jnp.zeros((16,), F32)
                s2 = jnp.zeros((16,), F32)
                for k in range(H // 16):
                    a = hi[e, pl.ds(16 * k, 16)]
                    bb = hj[e, pl.ds(16 * k, 16)]
                    d = jnp.abs(a - bb)
                    ob[e, pl.ds(16 * k, 16)] = d
                    s1 = s1 + d
                    s2 = s2 + d * d
                ecol = jnp.full((16,), e, jnp.int32)
                plsc.store_scatter(s1t, [iota, ecol], s1)
                plsc.store_scatter(s2t, [iota, ecol], s2)

            for g in range(3):
                S1 = s1t[0, pl.ds(16 * g, 16)]
                S2 = s2t[0, pl.ds(16 * g, 16)]
                for k in range(1, 16):
                    S1 = S1 + s1t[k, pl.ds(16 * g, 16)]
                    S2 = S2 + s2t[k, pl.ds(16 * g, 16)]
                viv = (S2 - S1 * S1 * (1.0 / H)) * (1.0 / (H - 1))
                rows = iota + (16 * g)
                m = None if g < 2 else halfmask
                plsc.store_scatter(ob, [rows, col_vi], viv, mask=m)
                plsc.store_scatter(ob, [rows, col_ct], ones16, mask=m)

        @pl.loop(0, BLK)
        def _block(blk):
            @pl.when(blk > 0)
            def _drain():
                wait_scatter(0, BCH - 2)
                wait_scatter(1, BCH - 1)

            pltpu.sync_copy(row_hbm.at[wid, blk], rb)
            pltpu.sync_copy(col_hbm.at[wid, blk], cb)
            for b in range(2):
                issue_gathers(b, b)

            @pl.loop(0, BCH // 2)
            def _pair(i):
                for b in range(2):
                    ch = 2 * i + b
                    wait_gathers(b, ch)

                    @pl.when(ch >= 2)
                    def _ws():
                        wait_scatter(b, ch - 2)

                    compute_chunk(b)
                    pltpu.async_copy(OB[b], acc.at[rb.at[ch]], SS[b],
                                     add=True)

                    @pl.when(i < (BCH // 2) - 1)
                    def _pref():
                        issue_gathers(b, ch + 2)

        wait_scatter(0, BCH - 2)
        wait_scatter(1, BCH - 1)
        plsc.subcore_barrier()
        pltpu.sync_copy(acc.at[pl.ds(s * RPT, RPT)],
                        out_hbm.at[c, pl.ds(s * RPT, RPT)])

    return edge_kernel


_EDGE_KERNEL = _sc_edge_kernel()


def _dot_t(a, w):
    return lax.dot_general(a, w, (((1,), (1,)), ((), ())),
                           preferred_element_type=F32)


BN = 1000


def _pre_body(x_ref, w_ref, b_ref, o_ref):
    o_ref[...] = jnp.maximum(
        _dot_t(x_ref[...], w_ref[...]) + b_ref[...], 0.0)


def _step_core(p0, p1, h, wih, whh, bih, bhh, tp):
    agg = p0 + p1
    diff = agg[:, :H]
    vm = agg[:, H:H + 1] / (agg[:, H + 1:H + 2] + 1e-6)
    t = jnp.maximum(vm * tp[0:1, :] + tp[1:2, :], 0.0)
    tau = jax.nn.sigmoid(
        jnp.sum(t * tp[2:3, :], axis=1, keepdims=True) + tp[3, 0])
    alpha = jax.nn.sigmoid(TAU_THRESH - tau)
    gi = _dot_t(diff, wih) + bih
    gh = _dot_t(h, whh) + bhh
    r = jax.nn.sigmoid(gi[:, :H] + gh[:, :H])
    z = jax.nn.sigmoid(gi[:, H:2 * H] + gh[:, H:2 * H])
    n = jnp.tanh(gi[:, 2 * H:] + r * gh[:, 2 * H:])
    nh = (1.0 - z) * n + z * h
    return alpha * nh + (1.0 - alpha) * h


def _step_body(p0_ref, p1_ref, h_ref, wih_ref, whh_ref, bih_ref, bhh_ref,
               tp_ref, o_ref):
    o_ref[...] = _step_core(p0_ref[...], p1_ref[...], h_ref[...],
                            wih_ref[...], whh_ref[...], bih_ref[...],
                            bhh_ref[...], tp_ref[...])


def _step_final_body(p0_ref, p1_ref, h_ref, wih_ref, whh_ref, bih_ref,
                     bhh_ref, tp_ref, wout_ref, bout_ref, o_ref):
    hn = _step_core(p0_ref[...], p1_ref[...], h_ref[...],
                    wih_ref[...], whh_ref[...], bih_ref[...],
                    bhh_ref[...], tp_ref[...])
    o_ref[...] = _dot_t(hn, wout_ref[...]) + bout_ref[...]


def _blk(shape):
    return pl.BlockSpec(shape, lambda i: (0,) * len(shape))


def _nblk(width):
    return pl.BlockSpec((BN, width), lambda i: (i, 0))


_PRE_CALL = pl.pallas_call(
    _pre_body,
    grid=(N // BN,),
    in_specs=[_nblk(H), _blk((H, H)), _blk((1, H))],
    out_specs=_nblk(H),
    out_shape=jax.ShapeDtypeStruct((N, H), F32),
)

_STEP_CALL = pl.pallas_call(
    _step_body,
    grid=(N // BN,),
    in_specs=[_nblk(AW), _nblk(AW), _nblk(H),
              _blk((3 * H, H)), _blk((3 * H, H)),
              _blk((1, 3 * H)), _blk((1, 3 * H)), _blk((4, 16))],
    out_specs=_nblk(H),
    out_shape=jax.ShapeDtypeStruct((N, H), F32),
)

_STEP_FINAL_CALL = pl.pallas_call(
    _step_final_body,
    grid=(N // BN,),
    in_specs=[_nblk(AW), _nblk(AW), _nblk(H),
              _blk((3 * H, H)), _blk((3 * H, H)),
              _blk((1, 3 * H)), _blk((1, 3 * H)), _blk((4, 16)),
              _blk((H, H)), _blk((1, H))],
    out_specs=_nblk(H),
    out_shape=jax.ShapeDtypeStruct((N, H), F32),
)


def kernel(x, edge_index, W_in, b_in, W_tau1, b_tau1, W_tau2, b_tau2,
           W_ih, W_hh, b_ih, b_hh, W_out, b_out):
    row = edge_index[0].reshape(NW, BLK, BCH, C)
    col = edge_index[1].reshape(NW, BLK, BCH, C)
    zeros = jnp.zeros((NP, AW), F32)
    bih = b_ih.reshape(1, 3 * H)
    bhh = b_hh.reshape(1, 3 * H)
    tp = jnp.stack([W_tau1[:, 0], b_tau1, W_tau2[0],
                    jnp.pad(b_tau2, (0, 15))])
    h = _PRE_CALL(x, W_in, b_in.reshape(1, H))
    for step in range(STEPS):
        parts = _EDGE_KERNEL(h, row, col, zeros)
        p0 = parts[0, :N]
        p1 = parts[1, :N]
        if step < STEPS - 1:
            h = _STEP_CALL(p0, p1, h, W_ih, W_hh, bih, bhh, tp)
        else:
            h = _STEP_FINAL_CALL(p0, p1, h, W_ih, W_hh, bih,
                                 bhh, tp, W_out, b_out.reshape(1, H))
    return h


# speedup vs baseline: 8.2746x; 1.9395x over previous
import functools

import jax
import jax.numpy as jnp
from jax import lax
from jax.experimental import pallas as pl
from jax.experimental.pallas import tpu as pltpu
from jax.experimental.pallas import tpu_sc as plsc

F32 = jnp.float32
STEPS = 3
TAU_THRESH = 0.05
N = 10000
E = 320000
H = 128
AW = H + 16
NC, NS = 2, 16
NW = NC * NS
EPW = E // NW
C = 40
BCH = 50
BLK = EPW // (BCH * C)
NP = 10240
RPT = NP // NS


def _sc_edge_kernel():
    mesh = plsc.VectorSubcoreMesh(core_axis_name="c", subcore_axis_name="s",
                                  num_cores=NC, num_subcores=NS)

    @functools.partial(
        pl.kernel,
        out_type=jax.ShapeDtypeStruct((NC, NP, AW), F32),
        mesh=mesh,
        compiler_params=pltpu.CompilerParams(needs_layout_passes=False,
                                             use_tc_tiling_on_sc=False),
        scratch_types=[
            pltpu.VMEM((BCH, C), jnp.int32),
            pltpu.VMEM((BCH, C), jnp.int32),
            pltpu.VMEM((C, H), F32),
            pltpu.VMEM((C, H), F32),
            pltpu.VMEM((C, H), F32),
            pltpu.VMEM((C, H), F32),
            pltpu.VMEM((C, AW), F32),
            pltpu.VMEM((C, AW), F32),
            pltpu.VMEM((16, 48), F32),
            pltpu.VMEM((16, 48), F32),
            pltpu.VMEM_SHARED((NP, AW), F32),
            pltpu.SemaphoreType.DMA,
            pltpu.SemaphoreType.DMA,
            pltpu.SemaphoreType.DMA,
            pltpu.SemaphoreType.DMA,
        ],
    )
    def edge_kernel(h_hbm, row_hbm, col_hbm, zeros_hbm, out_hbm,
                    rb, cb, hi0, hi1, hj0, hj1, ob0, ob1, s1t, s2t, acc,
                    sg0, sg1, ss0, ss1):
        c = lax.axis_index("c")
        s = lax.axis_index("s")
        wid = s * NC + c
        HI = (hi0, hi1)
        HJ = (hj0, hj1)
        OB = (ob0, ob1)
        SG = (sg0, sg1)
        SS = (ss0, ss1)

        iota = lax.iota(jnp.int32, 16)
        ones16 = jnp.full((16,), 1.0, F32)
        col_vi = jnp.full((16,), H, jnp.int32)
        col_ct = jnp.full((16,), H + 1, jnp.int32)
        halfmask = iota < 8

        pltpu.sync_copy(zeros_hbm.at[pl.ds(s * RPT, RPT)],
                        acc.at[pl.ds(s * RPT, RPT)])
        plsc.subcore_barrier()

        def issue_gathers(b, ch):
            cpi = pltpu.async_copy(h_hbm.at[rb.at[ch]], HI[b], SG[b])
            cpj = pltpu.async_copy(h_hbm.at[cb.at[ch]], HJ[b], SG[b])
            return cpi, cpj

        def wait_gathers(b, ch):
            pltpu.make_async_copy(h_hbm.at[rb.at[ch]], HI[b], SG[b]).wait()
            pltpu.make_async_copy(h_hbm.at[cb.at[ch]], HJ[b], SG[b]).wait()

        def wait_scatter(b, ch):
            pltpu.make_async_copy(OB[b], acc.at[rb.at[ch]], SS[b]).wait()

        def compute_chunk(b):
            hi, hj, ob = HI[b], HJ[b], OB[b]

            @plsc.parallel_loop(0, C, unroll=2)
            def _edge(e):
                s1 = jnp.zeros((16,), F32)
                s2 = jnp.zeros((16,), F32)
                for k in range(H // 16):
                    a = hi[e, pl.ds(16 * k, 16)]
                    bb = hj[e, pl.ds(16 * k, 16)]
                    d = jnp.abs(a - bb)
                    ob[e, pl.ds(16 * k, 16)] = d
                    s1 = s1 + d
                    s2 = s2 + d * d
                ecol = jnp.full((16,), e, jnp.int32)
                plsc.store_scatter(s1t, [iota, ecol], s1)
                plsc.store_scatter(s2t, [iota, ecol], s2)

            for g in range(3):
                S1 = s1t[0, pl.ds(16 * g, 16)]
                S2 = s2t[0, pl.ds(16 * g, 16)]
                for k in range(1, 16):
                    S1 = S1 + s1t[k, pl.ds(16 * g, 16)]
                    S2 = S2 + s2t[k, pl.ds(16 * g, 16)]
                viv = (S2 - S1 * S1 * (1.0 / H)) * (1.0 / (H - 1))
                rows = iota + (16 * g)
                m = None if g < 2 else halfmask
                plsc.store_scatter(ob, [rows, col_vi], viv, mask=m)
                plsc.store_scatter(ob, [rows, col_ct], ones16, mask=m)

        @pl.loop(0, BLK)
        def _block(blk):
            @pl.when(blk > 0)
            def _drain():
                wait_scatter(0, BCH - 2)
                wait_scatter(1, BCH - 1)

            pltpu.sync_copy(row_hbm.at[wid, blk], rb)
            pltpu.sync_copy(col_hbm.at[wid, blk], cb)
            for b in range(2):
                issue_gathers(b, b)

            @pl.loop(0, BCH // 2)
            def _pair(i):
                for b in range(2):
                    ch = 2 * i + b
                    wait_gathers(b, ch)

                    @pl.when(ch >= 2)
                    def _ws():
                        wait_scatter(b, ch - 2)

                    compute_chunk(b)
                    pltpu.async_copy(OB[b], acc.at[rb.at[ch]], SS[b],
                                     add=True)

                    @pl.when(i < (BCH // 2) - 1)
                    def _pref():
                        issue_gathers(b, ch + 2)

        wait_scatter(0, BCH - 2)
        wait_scatter(1, BCH - 1)
        plsc.subcore_barrier()
        pltpu.sync_copy(acc.at[pl.ds(s * RPT, RPT)],
                        out_hbm.at[c, pl.ds(s * RPT, RPT)])

    return edge_kernel


_EDGE_KERNEL = _sc_edge_kernel()


def _dot_t(a, w):
    return lax.dot_general(a, w, (((1,), (1,)), ((), ())),
                           preferred_element_type=F32)


BN = 1000


def _pre_body(x_ref, w_ref, b_ref, o_ref):
    o_ref[...] = jnp.maximum(
        _dot_t(x_ref[...], w_ref[...]) + b_ref[...], 0.0)


def _step_core(p0, p1, h, wih, whh, bih, bhh, tp):
    agg = p0 + p1
    diff = agg[:, :H]
    vm = agg[:, H:H + 1] / (agg[:, H + 1:H + 2] + 1e-6)
    t = jnp.maximum(vm * tp[0:1, :] + tp[1:2, :], 0.0)
    tau = jax.nn.sigmoid(
        jnp.sum(t * tp[2:3, :], axis=1, keepdims=True) + tp[3, 0])
    alpha = jax.nn.sigmoid(TAU_THRESH - tau)
    gi = _dot_t(diff, wih) + bih
    gh = _dot_t(h, whh) + bhh
    r = jax.nn.sigmoid(gi[:, :H] + gh[:, :H])
    z = jax.nn.sigmoid(gi[:, H:2 * H] + gh[:, H:2 * H])
    n = jnp.tanh(gi[:, 2 * H:] + r * gh[:, 2 * H:])
    nh = (1.0 - z) * n + z * h
    return alpha * nh + (1.0 - alpha) * h


def _step_body(p0_ref, p1_ref, h_ref, wih_ref, whh_ref, bih_ref, bhh_ref,
               tp_ref, o_ref):
    o_ref[...] = _step_core(p0_ref[...], p1_ref[...], h_ref[...],
                            wih_ref[...], whh_ref[...], bih_ref[...],
                            bhh_ref[...], tp_ref[...])


def _step_final_body(p0_ref, p1_ref, h_ref, wih_ref, whh_ref, bih_ref,
                     bhh_ref, tp_ref, wout_ref, bout_ref, o_ref):
    hn = _step_core(p0_ref[...], p1_ref[...], h_ref[...],
                    wih_ref[...], whh_ref[...], bih_ref[...],
                    bhh_ref[...], tp_ref[...])
    o_ref[...] = _dot_t(hn, wout_ref[...]) + bout_ref[...]


def _blk(shape):
    return pl.BlockSpec(shape, lambda i: (0,) * len(shape))


def _nblk(width):
    return pl.BlockSpec((BN, width), lambda i: (i, 0))


_PRE_CALL = pl.pallas_call(
    _pre_body,
    grid=(N // BN,),
    in_specs=[_nblk(H), _blk((H, H)), _blk((1, H))],
    out_specs=_nblk(H),
    out_shape=jax.ShapeDtypeStruct((N, H), F32),
)

_STEP_CALL = pl.pallas_call(
    _step_body,
    grid=(N // BN,),
    in_specs=[_nblk(AW), _nblk(AW), _nblk(H),
              _blk((3 * H, H)), _blk((3 * H, H)),
              _blk((1, 3 * H)), _blk((1, 3 * H)), _blk((4, 16))],
    out_specs=_nblk(H),
    out_shape=jax.ShapeDtypeStruct((N, H), F32),
)

_STEP_FINAL_CALL = pl.pallas_call(
    _step_final_body,
    grid=(N // BN,),
    in_specs=[_nblk(AW), _nblk(AW), _nblk(H),
              _blk((3 * H, H)), _blk((3 * H, H)),
              _blk((1, 3 * H)), _blk((1, 3 * H)), _blk((4, 16)),
              _blk((H, H)), _blk((1, H))],
    out_specs=_nblk(H),
    out_shape=jax.ShapeDtypeStruct((N, H), F32),
)


def kernel(x, edge_index, W_in, b_in, W_tau1, b_tau1, W_tau2, b_tau2,
           W_ih, W_hh, b_ih, b_hh, W_out, b_out):
    row = edge_index[0].reshape(NW, BLK, BCH, C)
    col = edge_index[1].reshape(NW, BLK, BCH, C)
    zeros = jnp.zeros((NP, AW), F32)
    bih = b_ih.reshape(1, 3 * H)
    bhh = b_hh.reshape(1, 3 * H)
    tp = jnp.stack([W_tau1[:, 0], b_tau1, W_tau2[0],
                    jnp.pad(b_tau2, (0, 15))])
    h = _PRE_CALL(x, W_in, b_in.reshape(1, H))
    for step in range(STEPS):
        parts = _EDGE_KERNEL(h, row, col, zeros)
        p0 = parts[0, :N]
        p1 = parts[1, :N]
        if step < STEPS - 1:
            h = _STEP_CALL(p0, p1, h, W_ih, W_hh, bih, bhh, tp)
        else:
            h = _STEP_FINAL_CALL(p0, p1, h, W_ih, W_hh, bih,
                                 bhh, tp, W_out, b_out.reshape(1, H))
    return h


# speedup vs baseline: 8.3217x; 1.0057x over previous
import functools

import jax
import jax.numpy as jnp
from jax import lax
from jax.experimental import pallas as pl
from jax.experimental.pallas import tpu as pltpu
from jax.experimental.pallas import tpu_sc as plsc

F32 = jnp.float32
STEPS = 3
TAU_THRESH = 0.05
N = 10000
E = 320000
H = 128
AW = H + 16
NC, NS = 2, 16
NW = NC * NS
EPW = E // NW
C = 40
BCH = 50
BLK = EPW // (BCH * C)
NP = 10240
RPT = NP // NS


def _sc_edge_kernel():
    mesh = plsc.VectorSubcoreMesh(core_axis_name="c", subcore_axis_name="s",
                                  num_cores=NC, num_subcores=NS)

    @functools.partial(
        pl.kernel,
        out_type=jax.ShapeDtypeStruct((NC, NP, AW), F32),
        mesh=mesh,
        compiler_params=pltpu.CompilerParams(needs_layout_passes=False,
                                             use_tc_tiling_on_sc=False),
        scratch_types=[
            pltpu.VMEM((BCH, C), jnp.int32),
            pltpu.VMEM((BCH, C), jnp.int32),
            pltpu.VMEM((C, H), F32),
            pltpu.VMEM((C, H), F32),
            pltpu.VMEM((C, H), F32),
            pltpu.VMEM((C, H), F32),
            pltpu.VMEM((C, AW), F32),
            pltpu.VMEM((C, AW), F32),
            pltpu.VMEM((16, 48), F32),
            pltpu.VMEM((16, 48), F32),
            pltpu.VMEM_SHARED((NP, AW), F32),
            pltpu.SemaphoreType.DMA,
            pltpu.SemaphoreType.DMA,
            pltpu.SemaphoreType.DMA,
            pltpu.SemaphoreType.DMA,
        ],
    )
    def edge_kernel(h_hbm, row_hbm, col_hbm, zeros_hbm, out_hbm,
                    rb, cb, hi0, hi1, hj0, hj1, ob0, ob1, s1t, s2t, acc,
                    sg0, sg1, ss0, ss1):
        c = lax.axis_index("c")
        s = lax.axis_index("s")
        wid = s * NC + c
        HI = (hi0, hi1)
        HJ = (hj0, hj1)
        OB = (ob0, ob1)
        SG = (sg0, sg1)
        SS = (ss0, ss1)

        iota = lax.iota(jnp.int32, 16)
        ones16 = jnp.full((16,), 1.0, F32)
        col_vi = jnp.full((16,), H, jnp.int32)
        col_ct = jnp.full((16,), H + 1, jnp.int32)
        halfmask = iota < 8

        pltpu.sync_copy(zeros_hbm.at[pl.ds(s * RPT, RPT)],
                        acc.at[pl.ds(s * RPT, RPT)])
        plsc.subcore_barrier()

        def issue_gathers(b, ch):
            cpi = pltpu.async_copy(h_hbm.at[rb.at[ch]], HI[b], SG[b])
            cpj = pltpu.async_copy(h_hbm.at[cb.at[ch]], HJ[b], SG[b])
            return cpi, cpj

        def wait_gathers(b, ch):
            pltpu.make_async_copy(h_hbm.at[rb.at[ch]], HI[b], SG[b]).wait()
            pltpu.make_async_copy(h_hbm.at[cb.at[ch]], HJ[b], SG[b]).wait()

        def wait_scatter(b, ch):
            pltpu.make_async_copy(OB[b], acc.at[rb.at[ch]], SS[b]).wait()

        def compute_chunk(b):
            hi, hj, ob = HI[b], HJ[b], OB[b]

            @plsc.parallel_loop(0, C, unroll=4)
            def _edge(e):
                s1 = jnp.zeros((16,), F32)
                s2 = jnp.zeros((16,), F32)
                for k in range(H // 16):
                    a = hi[e, pl.ds(16 * k, 16)]
                    bb = hj[e, pl.ds(16 * k, 16)]
                    d = jnp.abs(a - bb)
                    ob[e, pl.ds(16 * k, 16)] = d
                    s1 = s1 + d
                    s2 = s2 + d * d
                ecol = jnp.full((16,), e, jnp.int32)
                plsc.store_scatter(s1t, [iota, ecol], s1)
                plsc.store_scatter(s2t, [iota, ecol], s2)

            for g in range(3):
                S1 = s1t[0, pl.ds(16 * g, 16)]
                S2 = s2t[0, pl.ds(16 * g, 16)]
                for k in range(1, 16):
                    S1 = S1 + s1t[k, pl.ds(16 * g, 16)]
                    S2 = S2 + s2t[k, pl.ds(16 * g, 16)]
                viv = (S2 - S1 * S1 * (1.0 / H)) * (1.0 / (H - 1))
                rows = iota + (16 * g)
                m = None if g < 2 else halfmask
                plsc.store_scatter(ob, [rows, col_vi], viv, mask=m)
                plsc.store_scatter(ob, [rows, col_ct], ones16, mask=m)

        @pl.loop(0, BLK)
        def _block(blk):
            @pl.when(blk > 0)
            def _drain():
                wait_scatter(0, BCH - 2)
                wait_scatter(1, BCH - 1)

            pltpu.sync_copy(row_hbm.at[wid, blk], rb)
            pltpu.sync_copy(col_hbm.at[wid, blk], cb)
            for b in range(2):
                issue_gathers(b, b)

            @pl.loop(0, BCH // 2)
            def _pair(i):
                for b in range(2):
                    ch = 2 * i + b
                    wait_gathers(b, ch)

                    @pl.when(ch >= 2)
                    def _ws():
                        wait_scatter(b, ch - 2)

                    compute_chunk(b)
                    pltpu.async_copy(OB[b], acc.at[rb.at[ch]], SS[b],
                                     add=True)

                    @pl.when(i < (BCH // 2) - 1)
                    def _pref():
                        issue_gathers(b, ch + 2)

        wait_scatter(0, BCH - 2)
        wait_scatter(1, BCH - 1)
        plsc.subcore_barrier()
        pltpu.sync_copy(acc.at[pl.ds(s * RPT, RPT)],
                        out_hbm.at[c, pl.ds(s * RPT, RPT)])

    return edge_kernel


_EDGE_KERNEL = _sc_edge_kernel()


def _dot_t(a, w):
    return lax.dot_general(a, w, (((1,), (1,)), ((), ())),
                           preferred_element_type=F32)


BN = 1000


def _pre_body(x_ref, w_ref, b_ref, o_ref):
    o_ref[...] = jnp.maximum(
        _dot_t(x_ref[...], w_ref[...]) + b_ref[...], 0.0)


def _step_core(p0, p1, h, wih, whh, bih, bhh, tp):
    agg = p0 + p1
    diff = agg[:, :H]
    vm = agg[:, H:H + 1] / (agg[:, H + 1:H + 2] + 1e-6)
    t = jnp.maximum(vm * tp[0:1, :] + tp[1:2, :], 0.0)
    tau = jax.nn.sigmoid(
        jnp.sum(t * tp[2:3, :], axis=1, keepdims=True) + tp[3, 0])
    alpha = jax.nn.sigmoid(TAU_THRESH - tau)
    gi = _dot_t(diff, wih) + bih
    gh = _dot_t(h, whh) + bhh
    r = jax.nn.sigmoid(gi[:, :H] + gh[:, :H])
    z = jax.nn.sigmoid(gi[:, H:2 * H] + gh[:, H:2 * H])
    n = jnp.tanh(gi[:, 2 * H:] + r * gh[:, 2 * H:])
    nh = (1.0 - z) * n + z * h
    return alpha * nh + (1.0 - alpha) * h


def _step_body(p0_ref, p1_ref, h_ref, wih_ref, whh_ref, bih_ref, bhh_ref,
               tp_ref, o_ref):
    o_ref[...] = _step_core(p0_ref[...], p1_ref[...], h_ref[...],
                            wih_ref[...], whh_ref[...], bih_ref[...],
                            bhh_ref[...], tp_ref[...])


def _step_final_body(p0_ref, p1_ref, h_ref, wih_ref, whh_ref, bih_ref,
                     bhh_ref, tp_ref, wout_ref, bout_ref, o_ref):
    hn = _step_core(p0_ref[...], p1_ref[...], h_ref[...],
                    wih_ref[...], whh_ref[...], bih_ref[...],
                    bhh_ref[...], tp_ref[...])
    o_ref[...] = _dot_t(hn, wout_ref[...]) + bout_ref[...]


def _blk(shape):
    return pl.BlockSpec(shape, lambda i: (0,) * len(shape))


def _nblk(width):
    return pl.BlockSpec((BN, width), lambda i: (i, 0))


_PRE_CALL = pl.pallas_call(
    _pre_body,
    grid=(N // BN,),
    in_specs=[_nblk(H), _blk((H, H)), _blk((1, H))],
    out_specs=_nblk(H),
    out_shape=jax.ShapeDtypeStruct((N, H), F32),
)

_STEP_CALL = pl.pallas_call(
    _step_body,
    grid=(N // BN,),
    in_specs=[_nblk(AW), _nblk(AW), _nblk(H),
              _blk((3 * H, H)), _blk((3 * H, H)),
              _blk((1, 3 * H)), _blk((1, 3 * H)), _blk((4, 16))],
    out_specs=_nblk(H),
    out_shape=jax.ShapeDtypeStruct((N, H), F32),
)

_STEP_FINAL_CALL = pl.pallas_call(
    _step_final_body,
    grid=(N // BN,),
    in_specs=[_nblk(AW), _nblk(AW), _nblk(H),
              _blk((3 * H, H)), _blk((3 * H, H)),
              _blk((1, 3 * H)), _blk((1, 3 * H)), _blk((4, 16)),
              _blk((H, H)), _blk((1, H))],
    out_specs=_nblk(H),
    out_shape=jax.ShapeDtypeStruct((N, H), F32),
)


def kernel(x, edge_index, W_in, b_in, W_tau1, b_tau1, W_tau2, b_tau2,
           W_ih, W_hh, b_ih, b_hh, W_out, b_out):
    row = edge_index[0].reshape(NW, BLK, BCH, C)
    col = edge_index[1].reshape(NW, BLK, BCH, C)
    zeros = jnp.zeros((NP, AW), F32)
    bih = b_ih.reshape(1, 3 * H)
    bhh = b_hh.reshape(1, 3 * H)
    tp = jnp.stack([W_tau1[:, 0], b_tau1, W_tau2[0],
                    jnp.pad(b_tau2, (0, 15))])
    h = _PRE_CALL(x, W_in, b_in.reshape(1, H))
    for step in range(STEPS):
        parts = _EDGE_KERNEL(h, row, col, zeros)
        p0 = parts[0, :N]
        p1 = parts[1, :N]
        if step < STEPS - 1:
            h = _STEP_CALL(p0, p1, h, W_ih, W_hh, bih, bhh, tp)
        else:
            h = _STEP_FINAL_CALL(p0, p1, h, W_ih, W_hh, bih,
                                 bhh, tp, W_out, b_out.reshape(1, H))
    return h


# speedup vs baseline: 9.3338x; 1.1216x over previous
import functools

import jax
import jax.numpy as jnp
from jax import lax
from jax.experimental import pallas as pl
from jax.experimental.pallas import tpu as pltpu
from jax.experimental.pallas import tpu_sc as plsc

F32 = jnp.float32
STEPS = 3
TAU_THRESH = 0.05
N = 10000
E = 320000
H = 128
AW = H + 16
NC, NS = 2, 16
NW = NC * NS
EPW = E // NW
C = 40
BCH = 50
BLK = EPW // (BCH * C)
NP = 10240
RPT = NP // NS

import numpy as _np
_QPERM = _np.concatenate(
    [_np.concatenate([32 * k + 2 * _np.arange(16),
                      32 * k + 2 * _np.arange(16) + 1])
     for k in range(H // 32)])


def _sc_edge_kernel():
    mesh = plsc.VectorSubcoreMesh(core_axis_name="c", subcore_axis_name="s",
                                  num_cores=NC, num_subcores=NS)

    @functools.partial(
        pl.kernel,
        out_type=jax.ShapeDtypeStruct((NC, NP, AW), F32),
        mesh=mesh,
        compiler_params=pltpu.CompilerParams(needs_layout_passes=False,
                                             use_tc_tiling_on_sc=False),
        scratch_types=[
            pltpu.VMEM((BCH, C), jnp.int32),
            pltpu.VMEM((BCH, C), jnp.int32),
            pltpu.VMEM((C, H), jnp.bfloat16),
            pltpu.VMEM((C, H), jnp.bfloat16),
            pltpu.VMEM((C, H), jnp.bfloat16),
            pltpu.VMEM((C, H), jnp.bfloat16),
            pltpu.VMEM((C, AW), F32),
            pltpu.VMEM((C, AW), F32),
            pltpu.VMEM((16, 48), F32),
            pltpu.VMEM((16, 48), F32),
            pltpu.VMEM_SHARED((NP, AW), F32),
            pltpu.SemaphoreType.DMA,
            pltpu.SemaphoreType.DMA,
            pltpu.SemaphoreType.DMA,
            pltpu.SemaphoreType.DMA,
        ],
    )
    def edge_kernel(h_hbm, row_hbm, col_hbm, zeros_hbm, out_hbm,
                    rb, cb, hi0, hi1, hj0, hj1, ob0, ob1, s1t, s2t, acc,
                    sg0, sg1, ss0, ss1):
        c = lax.axis_index("c")
        s = lax.axis_index("s")
        wid = s * NC + c
        HI = (hi0, hi1)
        HJ = (hj0, hj1)
        OB = (ob0, ob1)
        SG = (sg0, sg1)
        SS = (ss0, ss1)

        iota = lax.iota(jnp.int32, 16)
        ones16 = jnp.full((16,), 1.0, F32)
        col_vi = jnp.full((16,), H, jnp.int32)
        col_ct = jnp.full((16,), H + 1, jnp.int32)
        halfmask = iota < 8

        pltpu.sync_copy(zeros_hbm.at[pl.ds(s * RPT, RPT)],
                        acc.at[pl.ds(s * RPT, RPT)])
        plsc.subcore_barrier()

        def issue_gathers(b, ch):
            cpi = pltpu.async_copy(h_hbm.at[rb.at[ch]], HI[b], SG[b])
            cpj = pltpu.async_copy(h_hbm.at[cb.at[ch]], HJ[b], SG[b])
            return cpi, cpj

        def wait_gathers(b, ch):
            pltpu.make_async_copy(h_hbm.at[rb.at[ch]], HI[b], SG[b]).wait()
            pltpu.make_async_copy(h_hbm.at[cb.at[ch]], HJ[b], SG[b]).wait()

        def wait_scatter(b, ch):
            pltpu.make_async_copy(OB[b], acc.at[rb.at[ch]], SS[b]).wait()

        def compute_chunk(b):
            hi, hj, ob = HI[b], HJ[b], OB[b]

            @plsc.parallel_loop(0, C, unroll=4)
            def _edge(e):
                s1 = jnp.zeros((16,), F32)
                s2 = jnp.zeros((16,), F32)
                for k in range(H // 32):
                    xa = hi[e, pl.ds(32 * k, 32)]
                    xb = hj[e, pl.ds(32 * k, 32)]
                    a1, a2 = plsc.unpack(xa, format=plsc.PackFormat.INTERLEAVED)
                    b1, b2 = plsc.unpack(xb, format=plsc.PackFormat.INTERLEAVED)
                    d1 = jnp.abs(a1 - b1)
                    d2 = jnp.abs(a2 - b2)
                    ob[e, pl.ds(32 * k, 16)] = d1
                    ob[e, pl.ds(32 * k + 16, 16)] = d2
                    s1 = s1 + d1 + d2
                    s2 = s2 + d1 * d1 + d2 * d2
                ecol = jnp.full((16,), e, jnp.int32)
                plsc.store_scatter(s1t, [iota, ecol], s1)
                plsc.store_scatter(s2t, [iota, ecol], s2)

            for g in range(3):
                S1 = s1t[0, pl.ds(16 * g, 16)]
                S2 = s2t[0, pl.ds(16 * g, 16)]
                for k in range(1, 16):
                    S1 = S1 + s1t[k, pl.ds(16 * g, 16)]
                    S2 = S2 + s2t[k, pl.ds(16 * g, 16)]
                viv = (S2 - S1 * S1 * (1.0 / H)) * (1.0 / (H - 1))
                rows = iota + (16 * g)
                m = None if g < 2 else halfmask
                plsc.store_scatter(ob, [rows, col_vi], viv, mask=m)
                plsc.store_scatter(ob, [rows, col_ct], ones16, mask=m)

        @pl.loop(0, BLK)
        def _block(blk):
            @pl.when(blk > 0)
            def _drain():
                wait_scatter(0, BCH - 2)
                wait_scatter(1, BCH - 1)

            pltpu.sync_copy(row_hbm.at[wid, blk], rb)
            pltpu.sync_copy(col_hbm.at[wid, blk], cb)
            for b in range(2):
                issue_gathers(b, b)

            @pl.loop(0, BCH // 2)
            def _pair(i):
                for b in range(2):
                    ch = 2 * i + b
                    wait_gathers(b, ch)

                    @pl.when(ch >= 2)
                    def _ws():
                        wait_scatter(b, ch - 2)

                    compute_chunk(b)
                    pltpu.async_copy(OB[b], acc.at[rb.at[ch]], SS[b],
                                     add=True)

                    @pl.when(i < (BCH // 2) - 1)
                    def _pref():
                        issue_gathers(b, ch + 2)

        wait_scatter(0, BCH - 2)
        wait_scatter(1, BCH - 1)
        plsc.subcore_barrier()
        pltpu.sync_copy(acc.at[pl.ds(s * RPT, RPT)],
                        out_hbm.at[c, pl.ds(s * RPT, RPT)])

    return edge_kernel


_EDGE_KERNEL = _sc_edge_kernel()


def _dot_t(a, w):
    return lax.dot_general(a, w, (((1,), (1,)), ((), ())),
                           preferred_element_type=F32)


BN = 1000


def _pre_body(x_ref, w_ref, b_ref, o_ref, obf_ref):
    h = jnp.maximum(_dot_t(x_ref[...], w_ref[...]) + b_ref[...], 0.0)
    o_ref[...] = h
    obf_ref[...] = h.astype(jnp.bfloat16)


def _step_core(p0, p1, h, wih, whh, bih, bhh, tp):
    agg = p0 + p1
    diff = agg[:, :H]
    vm = agg[:, H:H + 1] / (agg[:, H + 1:H + 2] + 1e-6)
    t = jnp.maximum(vm * tp[0:1, :] + tp[1:2, :], 0.0)
    tau = jax.nn.sigmoid(
        jnp.sum(t * tp[2:3, :], axis=1, keepdims=True) + tp[3, 0])
    alpha = jax.nn.sigmoid(TAU_THRESH - tau)
    gi = _dot_t(diff, wih) + bih
    gh = _dot_t(h, whh) + bhh
    r = jax.nn.sigmoid(gi[:, :H] + gh[:, :H])
    z = jax.nn.sigmoid(gi[:, H:2 * H] + gh[:, H:2 * H])
    n = jnp.tanh(gi[:, 2 * H:] + r * gh[:, 2 * H:])
    nh = (1.0 - z) * n + z * h
    return alpha * nh + (1.0 - alpha) * h


def _step_body(p0_ref, p1_ref, h_ref, wih_ref, whh_ref, bih_ref, bhh_ref,
               tp_ref, o_ref, obf_ref):
    h = _step_core(p0_ref[...], p1_ref[...], h_ref[...],
                   wih_ref[...], whh_ref[...], bih_ref[...],
                   bhh_ref[...], tp_ref[...])
    o_ref[...] = h
    obf_ref[...] = h.astype(jnp.bfloat16)


def _step_final_body(p0_ref, p1_ref, h_ref, wih_ref, whh_ref, bih_ref,
                     bhh_ref, tp_ref, wout_ref, bout_ref, o_ref):
    hn = _step_core(p0_ref[...], p1_ref[...], h_ref[...],
                    wih_ref[...], whh_ref[...], bih_ref[...],
                    bhh_ref[...], tp_ref[...])
    o_ref[...] = _dot_t(hn, wout_ref[...]) + bout_ref[...]


def _blk(shape):
    return pl.BlockSpec(shape, lambda i: (0,) * len(shape))


def _nblk(width):
    return pl.BlockSpec((BN, width), lambda i: (i, 0))


_PRE_CALL = pl.pallas_call(
    _pre_body,
    grid=(N // BN,),
    in_specs=[_nblk(H), _blk((H, H)), _blk((1, H))],
    out_specs=[_nblk(H), _nblk(H)],
    out_shape=[jax.ShapeDtypeStruct((N, H), F32),
               jax.ShapeDtypeStruct((N, H), jnp.bfloat16)],
)

_STEP_CALL = pl.pallas_call(
    _step_body,
    grid=(N // BN,),
    in_specs=[_nblk(AW), _nblk(AW), _nblk(H),
              _blk((3 * H, H)), _blk((3 * H, H)),
              _blk((1, 3 * H)), _blk((1, 3 * H)), _blk((4, 16))],
    out_specs=[_nblk(H), _nblk(H)],
    out_shape=[jax.ShapeDtypeStruct((N, H), F32),
               jax.ShapeDtypeStruct((N, H), jnp.bfloat16)],
)

_STEP_FINAL_CALL = pl.pallas_call(
    _step_final_body,
    grid=(N // BN,),
    in_specs=[_nblk(AW), _nblk(AW), _nblk(H),
              _blk((3 * H, H)), _blk((3 * H, H)),
              _blk((1, 3 * H)), _blk((1, 3 * H)), _blk((4, 16)),
              _blk((H, H)), _blk((1, H))],
    out_specs=_nblk(H),
    out_shape=jax.ShapeDtypeStruct((N, H), F32),
)


def kernel(x, edge_index, W_in, b_in, W_tau1, b_tau1, W_tau2, b_tau2,
           W_ih, W_hh, b_ih, b_hh, W_out, b_out):
    row = edge_index[0].reshape(NW, BLK, BCH, C)
    col = edge_index[1].reshape(NW, BLK, BCH, C)
    zeros = jnp.zeros((NP, AW), F32)
    bih = b_ih.reshape(1, 3 * H)
    bhh = b_hh.reshape(1, 3 * H)
    tp = jnp.stack([W_tau1[:, 0], b_tau1, W_tau2[0],
                    jnp.pad(b_tau2, (0, 15))])
    wih_q = W_ih[:, _QPERM]
    h, hbf = _PRE_CALL(x, W_in, b_in.reshape(1, H))
    for step in range(STEPS):
        parts = _EDGE_KERNEL(hbf, row, col, zeros)
        p0 = parts[0, :N]
        p1 = parts[1, :N]
        if step < STEPS - 1:
            h, hbf = _STEP_CALL(p0, p1, h, wih_q, W_hh, bih, bhh, tp)
        else:
            h = _STEP_FINAL_CALL(p0, p1, h, wih_q, W_hh, bih,
                                 bhh, tp, W_out, b_out.reshape(1, H))
    return h


# speedup vs baseline: 9.7571x; 1.0454x over previous
import functools

import jax
import jax.numpy as jnp
from jax import lax
from jax.experimental import pallas as pl
from jax.experimental.pallas import tpu as pltpu
from jax.experimental.pallas import tpu_sc as plsc

F32 = jnp.float32
STEPS = 3
TAU_THRESH = 0.05
N = 10000
E = 320000
H = 128
AW = H + 16
NC, NS = 2, 16
NW = NC * NS
EPW = E // NW
C = 40
BCH = 50
BLK = EPW // (BCH * C)
NP = 10240
RPT = NP // NS

import numpy as _np
_QPERM = _np.concatenate(
    [_np.concatenate([32 * k + 2 * _np.arange(16),
                      32 * k + 2 * _np.arange(16) + 1])
     for k in range(H // 32)])


def _sc_edge_kernel():
    mesh = plsc.VectorSubcoreMesh(core_axis_name="c", subcore_axis_name="s",
                                  num_cores=NC, num_subcores=NS)

    @functools.partial(
        pl.kernel,
        out_type=jax.ShapeDtypeStruct((NC, NP, AW), F32),
        mesh=mesh,
        compiler_params=pltpu.CompilerParams(needs_layout_passes=False,
                                             use_tc_tiling_on_sc=False),
        scratch_types=[
            pltpu.VMEM((BCH, C), jnp.int32),
            pltpu.VMEM((BCH, C), jnp.int32),
            pltpu.VMEM((C, H), jnp.bfloat16),
            pltpu.VMEM((C, H), jnp.bfloat16),
            pltpu.VMEM((C, H), jnp.bfloat16),
            pltpu.VMEM((C, H), jnp.bfloat16),
            pltpu.VMEM((C, AW), F32),
            pltpu.VMEM((C, AW), F32),
            pltpu.VMEM((16, 48), F32),
            pltpu.VMEM((16, 48), F32),
            pltpu.VMEM_SHARED((NP, AW), F32),
            pltpu.SemaphoreType.DMA,
            pltpu.SemaphoreType.DMA,
            pltpu.SemaphoreType.DMA,
            pltpu.SemaphoreType.DMA,
        ],
    )
    def edge_kernel(h_hbm, row_hbm, col_hbm, zeros_hbm, out_hbm,
                    rb, cb, hi0, hi1, hj0, hj1, ob0, ob1, s1t, s2t, acc,
                    sg0, sg1, ss0, ss1):
        c = lax.axis_index("c")
        s = lax.axis_index("s")
        wid = s * NC + c
        HI = (hi0, hi1)
        HJ = (hj0, hj1)
        OB = (ob0, ob1)
        SG = (sg0, sg1)
        SS = (ss0, ss1)

        iota = lax.iota(jnp.int32, 16)
        ones16 = jnp.full((16,), 1.0, F32)
        col_vi = jnp.full((16,), H, jnp.int32)
        col_ct = jnp.full((16,), H + 1, jnp.int32)
        halfmask = iota < 8

        pltpu.sync_copy(zeros_hbm, acc.at[pl.ds(s * RPT, RPT)])
        plsc.subcore_barrier()

        def issue_gathers(b, ch):
            cpi = pltpu.async_copy(h_hbm.at[rb.at[ch]], HI[b], SG[b])
            cpj = pltpu.async_copy(h_hbm.at[cb.at[ch]], HJ[b], SG[b])
            return cpi, cpj

        def wait_gathers(b, ch):
            pltpu.make_async_copy(h_hbm.at[rb.at[ch]], HI[b], SG[b]).wait()
            pltpu.make_async_copy(h_hbm.at[cb.at[ch]], HJ[b], SG[b]).wait()

        def wait_scatter(b, ch):
            pltpu.make_async_copy(OB[b], acc.at[rb.at[ch]], SS[b]).wait()

        def compute_chunk(b):
            hi, hj, ob = HI[b], HJ[b], OB[b]

            @plsc.parallel_loop(0, C, unroll=4)
            def _edge(e):
                s1 = jnp.zeros((16,), F32)
                s2 = jnp.zeros((16,), F32)
                for k in range(H // 32):
                    xa = hi[e, pl.ds(32 * k, 32)]
                    xb = hj[e, pl.ds(32 * k, 32)]
                    a1, a2 = plsc.unpack(xa, format=plsc.PackFormat.INTERLEAVED)
                    b1, b2 = plsc.unpack(xb, format=plsc.PackFormat.INTERLEAVED)
                    d1 = jnp.abs(a1 - b1)
                    d2 = jnp.abs(a2 - b2)
                    ob[e, pl.ds(32 * k, 16)] = d1
                    ob[e, pl.ds(32 * k + 16, 16)] = d2
                    s1 = s1 + d1 + d2
                    s2 = s2 + d1 * d1 + d2 * d2
                ecol = jnp.full((16,), e, jnp.int32)
                plsc.store_scatter(s1t, [iota, ecol], s1)
                plsc.store_scatter(s2t, [iota, ecol], s2)

            for g in range(3):
                S1 = s1t[0, pl.ds(16 * g, 16)]
                S2 = s2t[0, pl.ds(16 * g, 16)]
                for k in range(1, 16):
                    S1 = S1 + s1t[k, pl.ds(16 * g, 16)]
                    S2 = S2 + s2t[k, pl.ds(16 * g, 16)]
                viv = (S2 - S1 * S1 * (1.0 / H)) * (1.0 / (H - 1))
                rows = iota + (16 * g)
                m = None if g < 2 else halfmask
                plsc.store_scatter(ob, [rows, col_vi], viv, mask=m)
                plsc.store_scatter(ob, [rows, col_ct], ones16, mask=m)

        @pl.loop(0, BLK)
        def _block(blk):
            @pl.when(blk > 0)
            def _drain():
                wait_scatter(0, BCH - 2)
                wait_scatter(1, BCH - 1)

            pltpu.sync_copy(row_hbm.at[wid, blk], rb)
            pltpu.sync_copy(col_hbm.at[wid, blk], cb)
            for b in range(2):
                issue_gathers(b, b)

            @pl.loop(0, BCH // 2)
            def _pair(i):
                for b in range(2):
                    ch = 2 * i + b
                    wait_gathers(b, ch)

                    @pl.when(ch >= 2)
                    def _ws():
                        wait_scatter(b, ch - 2)

                    compute_chunk(b)
                    pltpu.async_copy(OB[b], acc.at[rb.at[ch]], SS[b],
                                     add=True)

                    @pl.when(i < (BCH // 2) - 1)
                    def _pref():
                        issue_gathers(b, ch + 2)

        wait_scatter(0, BCH - 2)
        wait_scatter(1, BCH - 1)
        plsc.subcore_barrier()
        pltpu.sync_copy(acc.at[pl.ds(s * RPT, RPT)],
                        out_hbm.at[c, pl.ds(s * RPT, RPT)])

    return edge_kernel


_EDGE_KERNEL = _sc_edge_kernel()


def _dot_t(a, w):
    return lax.dot_general(a, w, (((1,), (1,)), ((), ())),
                           preferred_element_type=F32)


BN = 1000


def _pre_body(x_ref, w_ref, b_ref, o_ref, obf_ref):
    h = jnp.maximum(_dot_t(x_ref[...], w_ref[...]) + b_ref[...], 0.0)
    o_ref[...] = h
    obf_ref[...] = h.astype(jnp.bfloat16)


def _step_core(parts, h, wih, whh, bih, bhh, tp):
    agg = parts[0] + parts[1]
    diff = agg[:, :H]
    vm = agg[:, H:H + 1] / (agg[:, H + 1:H + 2] + 1e-6)
    t = jnp.maximum(vm * tp[0:1, :] + tp[1:2, :], 0.0)
    tau = jax.nn.sigmoid(
        jnp.sum(t * tp[2:3, :], axis=1, keepdims=True) + tp[3, 0])
    alpha = jax.nn.sigmoid(TAU_THRESH - tau)
    gi = _dot_t(diff, wih) + bih
    gh = _dot_t(h, whh) + bhh
    r = jax.nn.sigmoid(gi[:, :H] + gh[:, :H])
    z = jax.nn.sigmoid(gi[:, H:2 * H] + gh[:, H:2 * H])
    n = jnp.tanh(gi[:, 2 * H:] + r * gh[:, 2 * H:])
    nh = (1.0 - z) * n + z * h
    return alpha * nh + (1.0 - alpha) * h


def _step_body(p_ref, h_ref, wih_ref, whh_ref, bih_ref, bhh_ref,
               tp_ref, o_ref, obf_ref):
    h = _step_core(p_ref[...], h_ref[...],
                   wih_ref[...], whh_ref[...], bih_ref[...],
                   bhh_ref[...], tp_ref[...])
    o_ref[...] = h
    obf_ref[...] = h.astype(jnp.bfloat16)


def _step_final_body(p_ref, h_ref, wih_ref, whh_ref, bih_ref,
                     bhh_ref, tp_ref, wout_ref, bout_ref, o_ref):
    hn = _step_core(p_ref[...], h_ref[...],
                    wih_ref[...], whh_ref[...], bih_ref[...],
                    bhh_ref[...], tp_ref[...])
    o_ref[...] = _dot_t(hn, wout_ref[...]) + bout_ref[...]


def _blk(shape):
    return pl.BlockSpec(shape, lambda i: (0,) * len(shape))


def _nblk(width):
    return pl.BlockSpec((BN, width), lambda i: (i, 0))


_PRE_CALL = pl.pallas_call(
    _pre_body,
    grid=(N // BN,),
    in_specs=[_nblk(H), _blk((H, H)), _blk((1, H))],
    out_specs=[_nblk(H), _nblk(H)],
    out_shape=[jax.ShapeDtypeStruct((N, H), F32),
               jax.ShapeDtypeStruct((N, H), jnp.bfloat16)],
)

_pblk = pl.BlockSpec((NC, BN, AW), lambda i: (0, i, 0))

_STEP_CALL = pl.pallas_call(
    _step_body,
    grid=(N // BN,),
    in_specs=[_pblk, _nblk(H),
              _blk((3 * H, H)), _blk((3 * H, H)),
              _blk((1, 3 * H)), _blk((1, 3 * H)), _blk((4, 16))],
    out_specs=[_nblk(H), _nblk(H)],
    out_shape=[jax.ShapeDtypeStruct((N, H), F32),
               jax.ShapeDtypeStruct((N, H), jnp.bfloat16)],
)

_STEP_FINAL_CALL = pl.pallas_call(
    _step_final_body,
    grid=(N // BN,),
    in_specs=[_pblk, _nblk(H),
              _blk((3 * H, H)), _blk((3 * H, H)),
              _blk((1, 3 * H)), _blk((1, 3 * H)), _blk((4, 16)),
              _blk((H, H)), _blk((1, H))],
    out_specs=_nblk(H),
    out_shape=jax.ShapeDtypeStruct((N, H), F32),
)


def kernel(x, edge_index, W_in, b_in, W_tau1, b_tau1, W_tau2, b_tau2,
           W_ih, W_hh, b_ih, b_hh, W_out, b_out):
    row = edge_index[0].reshape(NW, BLK, BCH, C)
    col = edge_index[1].reshape(NW, BLK, BCH, C)
    zeros = jnp.zeros((RPT, AW), F32)
    bih = b_ih.reshape(1, 3 * H)
    bhh = b_hh.reshape(1, 3 * H)
    tp = jnp.stack([W_tau1[:, 0], b_tau1, W_tau2[0],
                    jnp.pad(b_tau2, (0, 15))])
    wih_q = W_ih[:, _QPERM]
    h, hbf = _PRE_CALL(x, W_in, b_in.reshape(1, H))
    for step in range(STEPS):
        parts = _EDGE_KERNEL(hbf, row, col, zeros)
        if step < STEPS - 1:
            h, hbf = _STEP_CALL(parts, h, wih_q, W_hh, bih, bhh, tp)
        else:
            h = _STEP_FINAL_CALL(parts, h, wih_q, W_hh, bih,
                                 bhh, tp, W_out, b_out.reshape(1, H))
    return h


# speedup vs baseline: 9.7810x; 1.0024x over previous
import functools

import jax
import jax.numpy as jnp
from jax import lax
from jax.experimental import pallas as pl
from jax.experimental.pallas import tpu as pltpu
from jax.experimental.pallas import tpu_sc as plsc

F32 = jnp.float32
STEPS = 3
TAU_THRESH = 0.05
N = 10000
E = 320000
H = 128
AW = H + 16
NC, NS = 2, 16
NW = NC * NS
EPW = E // NW
C = 40
BCH = 50
BLK = EPW // (BCH * C)
NP = 10240
RPT = NP // NS

import numpy as _np
_QPERM = _np.concatenate(
    [_np.concatenate([32 * k + 2 * _np.arange(16),
                      32 * k + 2 * _np.arange(16) + 1])
     for k in range(H // 32)])


def _sc_edge_kernel():
    mesh = plsc.VectorSubcoreMesh(core_axis_name="c", subcore_axis_name="s",
                                  num_cores=NC, num_subcores=NS)

    @functools.partial(
        pl.kernel,
        out_type=jax.ShapeDtypeStruct((NC, NP, AW), F32),
        mesh=mesh,
        compiler_params=pltpu.CompilerParams(needs_layout_passes=False,
                                             use_tc_tiling_on_sc=False,
                                             skip_device_barrier=True),
        scratch_types=[
            pltpu.VMEM((BCH, C), jnp.int32),
            pltpu.VMEM((BCH, C), jnp.int32),
            pltpu.VMEM((C, H), jnp.bfloat16),
            pltpu.VMEM((C, H), jnp.bfloat16),
            pltpu.VMEM((C, H), jnp.bfloat16),
            pltpu.VMEM((C, H), jnp.bfloat16),
            pltpu.VMEM((C, AW), F32),
            pltpu.VMEM((C, AW), F32),
            pltpu.VMEM((16, 48), F32),
            pltpu.VMEM((16, 48), F32),
            pltpu.VMEM_SHARED((NP, AW), F32),
            pltpu.SemaphoreType.DMA,
            pltpu.SemaphoreType.DMA,
            pltpu.SemaphoreType.DMA,
            pltpu.SemaphoreType.DMA,
        ],
    )
    def edge_kernel(h_hbm, row_hbm, col_hbm, zeros_hbm, out_hbm,
                    rb, cb, hi0, hi1, hj0, hj1, ob0, ob1, s1t, s2t, acc,
                    sg0, sg1, ss0, ss1):
        c = lax.axis_index("c")
        s = lax.axis_index("s")
        wid = s * NC + c
        HI = (hi0, hi1)
        HJ = (hj0, hj1)
        OB = (ob0, ob1)
        SG = (sg0, sg1)
        SS = (ss0, ss1)

        iota = lax.iota(jnp.int32, 16)
        ones16 = jnp.full((16,), 1.0, F32)
        col_vi = jnp.full((16,), H, jnp.int32)
        col_ct = jnp.full((16,), H + 1, jnp.int32)
        halfmask = iota < 8

        def issue_gathers(b, ch):
            cpi = pltpu.async_copy(h_hbm.at[rb.at[ch]], HI[b], SG[b])
            cpj = pltpu.async_copy(h_hbm.at[cb.at[ch]], HJ[b], SG[b])
            return cpi, cpj

        def wait_gathers(b, ch):
            pltpu.make_async_copy(h_hbm.at[rb.at[ch]], HI[b], SG[b]).wait()
            pltpu.make_async_copy(h_hbm.at[cb.at[ch]], HJ[b], SG[b]).wait()

        def wait_scatter(b, ch):
            pltpu.make_async_copy(OB[b], acc.at[rb.at[ch]], SS[b]).wait()

        def compute_chunk(b):
            hi, hj, ob = HI[b], HJ[b], OB[b]

            @plsc.parallel_loop(0, C, unroll=4)
            def _edge(e):
                s1 = jnp.zeros((16,), F32)
                s2 = jnp.zeros((16,), F32)
                for k in range(H // 32):
                    xa = hi[e, pl.ds(32 * k, 32)]
                    xb = hj[e, pl.ds(32 * k, 32)]
                    a1, a2 = plsc.unpack(xa, format=plsc.PackFormat.INTERLEAVED)
                    b1, b2 = plsc.unpack(xb, format=plsc.PackFormat.INTERLEAVED)
                    d1 = jnp.abs(a1 - b1)
                    d2 = jnp.abs(a2 - b2)
                    ob[e, pl.ds(32 * k, 16)] = d1
                    ob[e, pl.ds(32 * k + 16, 16)] = d2
                    s1 = s1 + d1 + d2
                    s2 = s2 + d1 * d1 + d2 * d2
                ecol = jnp.full((16,), e, jnp.int32)
                plsc.store_scatter(s1t, [iota, ecol], s1)
                plsc.store_scatter(s2t, [iota, ecol], s2)

            for g in range(3):
                S1 = s1t[0, pl.ds(16 * g, 16)]
                S2 = s2t[0, pl.ds(16 * g, 16)]
                for k in range(1, 16):
                    S1 = S1 + s1t[k, pl.ds(16 * g, 16)]
                    S2 = S2 + s2t[k, pl.ds(16 * g, 16)]
                viv = (S2 - S1 * S1 * (1.0 / H)) * (1.0 / (H - 1))
                rows = iota + (16 * g)
                m = None if g < 2 else halfmask
                plsc.store_scatter(ob, [rows, col_vi], viv, mask=m)
                plsc.store_scatter(ob, [rows, col_ct], ones16, mask=m)

        @pl.loop(0, BLK)
        def _block(blk):
            @pl.when(blk > 0)
            def _drain():
                wait_scatter(0, BCH - 2)
                wait_scatter(1, BCH - 1)

            pltpu.sync_copy(row_hbm.at[wid, blk], rb)
            pltpu.sync_copy(col_hbm.at[wid, blk], cb)
            for b in range(2):
                issue_gathers(b, b)

            @pl.when(blk == 0)
            def _zero():
                pltpu.sync_copy(zeros_hbm, acc.at[pl.ds(s * RPT, RPT)])
                plsc.subcore_barrier()

            @pl.loop(0, BCH // 2)
            def _pair(i):
                for b in range(2):
                    ch = 2 * i + b
                    wait_gathers(b, ch)

                    @pl.when(ch >= 2)
                    def _ws():
                        wait_scatter(b, ch - 2)

                    compute_chunk(b)
                    pltpu.async_copy(OB[b], acc.at[rb.at[ch]], SS[b],
                                     add=True)

                    @pl.when(i < (BCH // 2) - 1)
                    def _pref():
                        issue_gathers(b, ch + 2)

        wait_scatter(0, BCH - 2)
        wait_scatter(1, BCH - 1)
        plsc.subcore_barrier()
        pltpu.sync_copy(acc.at[pl.ds(s * RPT, RPT)],
                        out_hbm.at[c, pl.ds(s * RPT, RPT)])

    return edge_kernel


_EDGE_KERNEL = _sc_edge_kernel()


def _dot_t(a, w):
    return lax.dot_general(a, w, (((1,), (1,)), ((), ())),
                           preferred_element_type=F32)


BN = 1000


def _pre_body(x_ref, w_ref, b_ref, o_ref, obf_ref):
    h = jnp.maximum(_dot_t(x_ref[...], w_ref[...]) + b_ref[...], 0.0)
    o_ref[...] = h
    obf_ref[...] = h.astype(jnp.bfloat16)


def _step_core(parts, h, wih, whh, bih, bhh, tp):
    agg = parts[0] + parts[1]
    diff = agg[:, :H]
    vm = agg[:, H:H + 1] / (agg[:, H + 1:H + 2] + 1e-6)
    t = jnp.maximum(vm * tp[0:1, :] + tp[1:2, :], 0.0)
    tau = jax.nn.sigmoid(
        jnp.sum(t * tp[2:3, :], axis=1, keepdims=True) + tp[3, 0])
    alpha = jax.nn.sigmoid(TAU_THRESH - tau)
    gi = _dot_t(diff, wih) + bih
    gh = _dot_t(h, whh) + bhh
    r = jax.nn.sigmoid(gi[:, :H] + gh[:, :H])
    z = jax.nn.sigmoid(gi[:, H:2 * H] + gh[:, H:2 * H])
    n = jnp.tanh(gi[:, 2 * H:] + r * gh[:, 2 * H:])
    nh = (1.0 - z) * n + z * h
    return alpha * nh + (1.0 - alpha) * h


def _step_body(p_ref, h_ref, wih_ref, whh_ref, bih_ref, bhh_ref,
               tp_ref, o_ref, obf_ref):
    h = _step_core(p_ref[...], h_ref[...],
                   wih_ref[...], whh_ref[...], bih_ref[...],
                   bhh_ref[...], tp_ref[...])
    o_ref[...] = h
    obf_ref[...] = h.astype(jnp.bfloat16)


def _step_final_body(p_ref, h_ref, wih_ref, whh_ref, bih_ref,
                     bhh_ref, tp_ref, wout_ref, bout_ref, o_ref):
    hn = _step_core(p_ref[...], h_ref[...],
                    wih_ref[...], whh_ref[...], bih_ref[...],
                    bhh_ref[...], tp_ref[...])
    o_ref[...] = _dot_t(hn, wout_ref[...]) + bout_ref[...]


def _blk(shape):
    return pl.BlockSpec(shape, lambda i: (0,) * len(shape))


def _nblk(width):
    return pl.BlockSpec((BN, width), lambda i: (i, 0))


_PRE_CALL = pl.pallas_call(
    _pre_body,
    grid=(N // BN,),
    in_specs=[_nblk(H), _blk((H, H)), _blk((1, H))],
    out_specs=[_nblk(H), _nblk(H)],
    out_shape=[jax.ShapeDtypeStruct((N, H), F32),
               jax.ShapeDtypeStruct((N, H), jnp.bfloat16)],
)

_pblk = pl.BlockSpec((NC, BN, AW), lambda i: (0, i, 0))

_STEP_CALL = pl.pallas_call(
    _step_body,
    grid=(N // BN,),
    in_specs=[_pblk, _nblk(H),
              _blk((3 * H, H)), _blk((3 * H, H)),
              _blk((1, 3 * H)), _blk((1, 3 * H)), _blk((4, 16))],
    out_specs=[_nblk(H), _nblk(H)],
    out_shape=[jax.ShapeDtypeStruct((N, H), F32),
               jax.ShapeDtypeStruct((N, H), jnp.bfloat16)],
)

_STEP_FINAL_CALL = pl.pallas_call(
    _step_final_body,
    grid=(N // BN,),
    in_specs=[_pblk, _nblk(H),
              _blk((3 * H, H)), _blk((3 * H, H)),
              _blk((1, 3 * H)), _blk((1, 3 * H)), _blk((4, 16)),
              _blk((H, H)), _blk((1, H))],
    out_specs=_nblk(H),
    out_shape=jax.ShapeDtypeStruct((N, H), F32),
)


def kernel(x, edge_index, W_in, b_in, W_tau1, b_tau1, W_tau2, b_tau2,
           W_ih, W_hh, b_ih, b_hh, W_out, b_out):
    row = edge_index[0].reshape(NW, BLK, BCH, C)
    col = edge_index[1].reshape(NW, BLK, BCH, C)
    zeros = jnp.zeros((RPT, AW), F32)
    bih = b_ih.reshape(1, 3 * H)
    bhh = b_hh.reshape(1, 3 * H)
    tp = jnp.stack([W_tau1[:, 0], b_tau1, W_tau2[0],
                    jnp.pad(b_tau2, (0, 15))])
    wih_q = W_ih[:, _QPERM]
    h, hbf = _PRE_CALL(x, W_in, b_in.reshape(1, H))
    for step in range(STEPS):
        parts = _EDGE_KERNEL(hbf, row, col, zeros)
        if step < STEPS - 1:
            h, hbf = _STEP_CALL(parts, h, wih_q, W_hh, bih, bhh, tp)
        else:
            h = _STEP_FINAL_CALL(parts, h, wih_q, W_hh, bih,
                                 bhh, tp, W_out, b_out.reshape(1, H))
    return h
